# baseline (speedup 1.0000x reference)
import functools

import jax
import jax.numpy as jnp
from jax import lax
from jax.experimental import pallas as pl
from jax.experimental.pallas import tpu as pltpu
from jax.experimental.pallas import tpu_sc as plsc

B, M, H = 8, 2048, 1024
T = B * M
LOGIT_THRESHOLD = 4.59511985013459

NC, NS, L = 2, 16, 16
NW = NC * NS
WPR = NW // B
PW = M // WPR
CHUNK = 32
NCH = PW // CHUNK


def _logits_tc(h_flat, W, b):
    blk = 512

    def body(h_ref, w_ref, b_ref, o_ref):
        o_ref[...] = (
            jnp.dot(h_ref[...], w_ref[...], preferred_element_type=jnp.float32)
            + b_ref[0]
        )

    return pl.pallas_call(
        body,
        grid=(T // blk,),
        in_specs=[
            pl.BlockSpec((blk, H), lambda i: (i, 0)),
            pl.BlockSpec((H, 1), lambda i: (0, 0)),
            pl.BlockSpec(memory_space=pltpu.SMEM),
        ],
        out_specs=pl.BlockSpec((blk, 1), lambda i: (i, 0)),
        out_shape=jax.ShapeDtypeStruct((T, 1), jnp.float32),
    )(h_flat, W, b)


def _pack_sc(h_flat, logits):
    mesh = plsc.VectorSubcoreMesh(
        core_axis_name="c", subcore_axis_name="s", num_cores=NC, num_subcores=NS
    )

    @functools.partial(
        pl.kernel,
        out_type=jax.ShapeDtypeStruct((T, H), jnp.float32),
        mesh=mesh,
        scratch_types=[
            pltpu.VMEM((M,), jnp.float32),
            pltpu.VMEM((M + L,), jnp.int32),
            pltpu.VMEM((CHUNK,), jnp.int32),
            pltpu.VMEM((CHUNK,), jnp.int32),
            pltpu.VMEM((CHUNK, H), jnp.float32),
            pltpu.VMEM((CHUNK, H), jnp.float32),
            pltpu.VMEM((H,), jnp.float32),
            pltpu.SemaphoreType.DMA,
            pltpu.SemaphoreType.DMA,
        ],
        compiler_params=pltpu.CompilerParams(needs_layout_passes=False),
    )
    def k(h_hbm, lg_hbm, out_hbm, lg_v, src_v, idx0, idx1, hbuf0, hbuf1, zrow, sem0, sem1):
        c = lax.axis_index("c")
        s = lax.axis_index("s")
        row = c * (B // NC) + s // WPR
        q = s % WPR
        row_base = row * M
        base = q * PW

        zf = jnp.zeros((L,), jnp.float32)
        zi = jnp.zeros((L,), jnp.int32)

        def zero_zrow(i, _):
            zrow[pl.ds(i * L, L)] = zf
            return 0

        lax.fori_loop(0, H // L, zero_zrow, 0)

        def zero_src(i, _):
            src_v[pl.ds(i * L, L)] = zi
            return 0

        lax.fori_loop(0, (M + L) // L, zero_src, 0)

        pltpu.sync_copy(lg_hbm.at[pl.ds(row_base, M)], lg_v)

        thresh = jnp.full((L,), LOGIT_THRESHOLD, jnp.float32)
        m_splat = jnp.full((L,), M, jnp.int32)

        def build(g, carry):
            running, fmin = carry
            lv = lg_v[pl.ds(g * L, L)]
            m = lv < thresh
            ids = lax.iota(jnp.int32, L) + g * L
            pos = running + plsc.cumsum(jnp.where(m, 1, 0).astype(jnp.int32)) - 1
            plsc.store_scatter(src_v, [pos], ids, mask=m)
            fmin = jnp.minimum(fmin, jnp.where(m, m_splat, ids))
            return running + plsc.all_reduce_population_count(m), fmin

        cnt_splat, f_splat = lax.fori_loop(
            0, M // L, build, (jnp.zeros((L,), jnp.int32), m_splat)
        )
        count = jnp.max(cnt_splat)
        first_halt = jnp.min(f_splat)

        @pl.when(first_halt >= M)
        def _():
            pltpu.sync_copy(
                h_hbm.at[pl.ds(row_base + base, PW)],
                out_hbm.at[pl.ds(row_base + base, PW)],
            )

        @pl.when(first_halt < M)
        def _():
            idxb = (idx0, idx1)
            hb = (hbuf0, hbuf1)
            sems = (sem0, sem1)

            def fill_issue(g):
                bu = g % 2
                j0 = base + g * CHUNK
                idxb[bu][pl.ds(0, L)] = src_v[pl.ds(j0, L)] + row_base
                idxb[bu][pl.ds(L, L)] = src_v[pl.ds(j0 + L, L)] + row_base
                return pltpu.async_copy(h_hbm.at[idxb[bu]], hb[bu], sems[bu])

            cops = [None, None]
            cops[0] = fill_issue(0)
            for g in range(NCH):
                bu = g % 2
                if g + 1 < NCH:
                    cops[1 - bu] = fill_issue(g + 1)
                cops[bu].wait()
                pltpu.sync_copy(
                    hb[bu], out_hbm.at[pl.ds(row_base + base + g * CHUNK, CHUNK)]
                )

            lo = jnp.clip(count, base, base + PW)

            def ztail(j, _):
                pltpu.sync_copy(zrow, out_hbm.at[row_base + j])
                return 0

            lax.fori_loop(lo, base + PW, ztail, 0)

    return k(h_flat, logits)


def kernel(h, W, b):
    h_flat = h.reshape(T, H)
    logits = _logits_tc(h_flat, W, b).reshape(T)
    out = _pack_sc(h_flat, logits)
    return out.reshape(B, M, H)


# speedup vs baseline: 32.4957x; 32.4957x over previous
import functools

import jax
import jax.numpy as jnp
from jax import lax
from jax.experimental import pallas as pl
from jax.experimental.pallas import tpu as pltpu
from jax.experimental.pallas import tpu_sc as plsc

B, M, H = 8, 2048, 1024
T = B * M
LOGIT_THRESHOLD = 4.59511985013459

NC, NS, L = 2, 16, 16
NW = NC * NS
WPR = NW // B
PW = M // WPR
CHUNK = 32
NCH = PW // CHUNK


def _logits_copy_tc(h_flat, W, b):
    blk = 2048

    def body(h_ref, w_ref, b_ref, o_ref, lg_ref, fl_ref):
        i = pl.program_id(0)
        blk_h = h_ref[...]
        o_ref[...] = blk_h
        lg = (
            jnp.dot(blk_h, w_ref[...], preferred_element_type=jnp.float32)
            + b_ref[0]
        )
        lg_ref[...] = lg
        cnt = jnp.sum((lg >= LOGIT_THRESHOLD).astype(jnp.float32))

        @pl.when(i == 0)
        def _():
            fl_ref[...] = jnp.zeros((8, 128), jnp.float32)

        fl_ref[...] = fl_ref[...] + jnp.full((8, 128), cnt)

    return pl.pallas_call(
        body,
        grid=(T // blk,),
        in_specs=[
            pl.BlockSpec((blk, H), lambda i: (i, 0)),
            pl.BlockSpec((H, 1), lambda i: (0, 0)),
            pl.BlockSpec(memory_space=pltpu.SMEM),
        ],
        out_specs=[
            pl.BlockSpec((blk, H), lambda i: (i, 0)),
            pl.BlockSpec((blk, 1), lambda i: (i, 0)),
            pl.BlockSpec((8, 128), lambda i: (0, 0)),
        ],
        out_shape=[
            jax.ShapeDtypeStruct((T, H), jnp.float32),
            jax.ShapeDtypeStruct((T, 1), jnp.float32),
            jax.ShapeDtypeStruct((8, 128), jnp.float32),
        ],
    )(h_flat, W, b)


def _pack_sc(h_flat, logits):
    mesh = plsc.VectorSubcoreMesh(
        core_axis_name="c", subcore_axis_name="s", num_cores=NC, num_subcores=NS
    )

    @functools.partial(
        pl.kernel,
        out_type=jax.ShapeDtypeStruct((T, H), jnp.float32),
        mesh=mesh,
        scratch_types=[
            pltpu.VMEM((M,), jnp.float32),
            pltpu.VMEM((M + L,), jnp.int32),
            pltpu.VMEM((CHUNK,), jnp.int32),
            pltpu.VMEM((CHUNK,), jnp.int32),
            pltpu.VMEM((CHUNK, H), jnp.float32),
            pltpu.VMEM((CHUNK, H), jnp.float32),
            pltpu.VMEM((H,), jnp.float32),
            pltpu.SemaphoreType.DMA,
            pltpu.SemaphoreType.DMA,
        ],
        compiler_params=pltpu.CompilerParams(needs_layout_passes=False),
    )
    def k(h_hbm, lg_hbm, out_hbm, lg_v, src_v, idx0, idx1, hbuf0, hbuf1, zrow, sem0, sem1):
        c = lax.axis_index("c")
        s = lax.axis_index("s")
        row = c * (B // NC) + s // WPR
        q = s % WPR
        row_base = row * M
        base = q * PW

        zf = jnp.zeros((L,), jnp.float32)
        zi = jnp.zeros((L,), jnp.int32)

        def zero_zrow(i, _):
            zrow[pl.ds(i * L, L)] = zf
            return 0

        lax.fori_loop(0, H // L, zero_zrow, 0)

        def zero_src(i, _):
            src_v[pl.ds(i * L, L)] = zi
            return 0

        lax.fori_loop(0, (M + L) // L, zero_src, 0)

        pltpu.sync_copy(lg_hbm.at[pl.ds(row_base, M)], lg_v)

        thresh = jnp.full((L,), LOGIT_THRESHOLD, jnp.float32)

        def build(g, running):
            lv = lg_v[pl.ds(g * L, L)]
            m = lv < thresh
            ids = lax.iota(jnp.int32, L) + g * L
            pos = running + plsc.cumsum(jnp.where(m, 1, 0).astype(jnp.int32)) - 1
            plsc.store_scatter(src_v, [pos], ids, mask=m)
            return running + plsc.all_reduce_population_count(m)

        cnt_splat = lax.fori_loop(0, M // L, build, jnp.zeros((L,), jnp.int32))
        count = jnp.max(cnt_splat)

        idxb = (idx0, idx1)
        hb = (hbuf0, hbuf1)
        sems = (sem0, sem1)

        def fill_issue(g):
            bu = g % 2
            j0 = base + g * CHUNK
            idxb[bu][pl.ds(0, L)] = src_v[pl.ds(j0, L)] + row_base
            idxb[bu][pl.ds(L, L)] = src_v[pl.ds(j0 + L, L)] + row_base
            return pltpu.async_copy(h_hbm.at[idxb[bu]], hb[bu], sems[bu])

        cops = [None, None]
        cops[0] = fill_issue(0)
        for g in range(NCH):
            bu = g % 2
            if g + 1 < NCH:
                cops[1 - bu] = fill_issue(g + 1)
            cops[bu].wait()
            pltpu.sync_copy(
                hb[bu], out_hbm.at[pl.ds(row_base + base + g * CHUNK, CHUNK)]
            )

        lo = jnp.clip(count, base, base + PW)

        def ztail(j, _):
            pltpu.sync_copy(zrow, out_hbm.at[row_base + j])
            return 0

        lax.fori_loop(lo, base + PW, ztail, 0)

    return k(h_flat, logits)


def kernel(h, W, b):
    h_flat = h.reshape(T, H)
    out_spec, logits, flags = _logits_copy_tc(h_flat, W, b)
    any_halt = flags[0, 0] > 0.5
    out = lax.cond(
        any_halt,
        lambda: _pack_sc(h_flat, logits.reshape(T)),
        lambda: out_spec,
    )
    return out.reshape(B, M, H)


# speedup vs baseline: 33.3374x; 1.0259x over previous
import functools

import jax
import jax.numpy as jnp
from jax import lax
from jax.experimental import pallas as pl
from jax.experimental.pallas import tpu as pltpu
from jax.experimental.pallas import tpu_sc as plsc

B, M, H = 8, 2048, 1024
T = B * M
LOGIT_THRESHOLD = 4.59511985013459

NC, NS, L = 2, 16, 16
NW = NC * NS
WPR = NW // B
PW = M // WPR
CHUNK = 32
NCH = PW // CHUNK


def _logits_copy_tc(h_flat, W, b):
    blk = 2048

    def body(h_ref, w_ref, b_ref, o_ref, lg_ref, fl_ref):
        i = pl.program_id(0)
        blk_h = h_ref[...]
        o_ref[...] = blk_h
        lg = (
            jnp.dot(blk_h, w_ref[...], preferred_element_type=jnp.float32)
            + b_ref[0]
        )
        lg_ref[...] = lg
        cnt = jnp.sum((lg >= LOGIT_THRESHOLD).astype(jnp.float32))

        @pl.when(i == 0)
        def _():
            fl_ref[0] = 0.0

        fl_ref[0] = fl_ref[0] + cnt

    return pl.pallas_call(
        body,
        grid=(T // blk,),
        in_specs=[
            pl.BlockSpec((blk, H), lambda i: (i, 0)),
            pl.BlockSpec((H, 1), lambda i: (0, 0)),
            pl.BlockSpec(memory_space=pltpu.SMEM),
        ],
        out_specs=[
            pl.BlockSpec((blk, H), lambda i: (i, 0)),
            pl.BlockSpec((blk, 1), lambda i: (i, 0)),
            pl.BlockSpec(memory_space=pltpu.SMEM),
        ],
        out_shape=[
            jax.ShapeDtypeStruct((T, H), jnp.float32),
            jax.ShapeDtypeStruct((T, 1), jnp.float32),
            jax.ShapeDtypeStruct((1,), jnp.float32),
        ],
    )(h_flat, W, b)


def _pack_sc(h_flat, logits):
    mesh = plsc.VectorSubcoreMesh(
        core_axis_name="c", subcore_axis_name="s", num_cores=NC, num_subcores=NS
    )

    @functools.partial(
        pl.kernel,
        out_type=jax.ShapeDtypeStruct((T, H), jnp.float32),
        mesh=mesh,
        scratch_types=[
            pltpu.VMEM((M,), jnp.float32),
            pltpu.VMEM((M + L,), jnp.int32),
            pltpu.VMEM((CHUNK,), jnp.int32),
            pltpu.VMEM((CHUNK,), jnp.int32),
            pltpu.VMEM((CHUNK, H), jnp.float32),
            pltpu.VMEM((CHUNK, H), jnp.float32),
            pltpu.VMEM((H,), jnp.float32),
            pltpu.SemaphoreType.DMA,
            pltpu.SemaphoreType.DMA,
        ],
        compiler_params=pltpu.CompilerParams(needs_layout_passes=False),
    )
    def k(h_hbm, lg_hbm, out_hbm, lg_v, src_v, idx0, idx1, hbuf0, hbuf1, zrow, sem0, sem1):
        c = lax.axis_index("c")
        s = lax.axis_index("s")
        row = c * (B // NC) + s // WPR
        q = s % WPR
        row_base = row * M
        base = q * PW

        zf = jnp.zeros((L,), jnp.float32)
        zi = jnp.zeros((L,), jnp.int32)

        def zero_zrow(i, _):
            zrow[pl.ds(i * L, L)] = zf
            return 0

        lax.fori_loop(0, H // L, zero_zrow, 0)

        def zero_src(i, _):
            src_v[pl.ds(i * L, L)] = zi
            return 0

        lax.fori_loop(0, (M + L) // L, zero_src, 0)

        pltpu.sync_copy(lg_hbm.at[pl.ds(row_base, M)], lg_v)

        thresh = jnp.full((L,), LOGIT_THRESHOLD, jnp.float32)

        def build(g, running):
            lv = lg_v[pl.ds(g * L, L)]
            m = lv < thresh
            ids = lax.iota(jnp.int32, L) + g * L
            pos = running + plsc.cumsum(jnp.where(m, 1, 0).astype(jnp.int32)) - 1
            plsc.store_scatter(src_v, [pos], ids, mask=m)
            return running + plsc.all_reduce_population_count(m)

        cnt_splat = lax.fori_loop(0, M // L, build, jnp.zeros((L,), jnp.int32))
        count = jnp.max(cnt_splat)

        idxb = (idx0, idx1)
        hb = (hbuf0, hbuf1)
        sems = (sem0, sem1)

        def fill_issue(g):
            bu = g % 2
            j0 = base + g * CHUNK
            idxb[bu][pl.ds(0, L)] = src_v[pl.ds(j0, L)] + row_base
            idxb[bu][pl.ds(L, L)] = src_v[pl.ds(j0 + L, L)] + row_base
            return pltpu.async_copy(h_hbm.at[idxb[bu]], hb[bu], sems[bu])

        cops = [None, None]
        cops[0] = fill_issue(0)
        for g in range(NCH):
            bu = g % 2
            if g + 1 < NCH:
                cops[1 - bu] = fill_issue(g + 1)
            cops[bu].wait()
            pltpu.sync_copy(
                hb[bu], out_hbm.at[pl.ds(row_base + base + g * CHUNK, CHUNK)]
            )

        lo = jnp.clip(count, base, base + PW)

        def ztail(j, _):
            pltpu.sync_copy(zrow, out_hbm.at[row_base + j])
            return 0

        lax.fori_loop(lo, base + PW, ztail, 0)

    return k(h_flat, logits)


def kernel(h, W, b):
    h_flat = h.reshape(T, H)
    out_spec, logits, flags = _logits_copy_tc(h_flat, W, b)
    any_halt = flags[0] > 0.5
    out = lax.cond(
        any_halt,
        lambda: _pack_sc(h_flat, logits.reshape(T)),
        lambda: out_spec,
    )
    return out.reshape(B, M, H)
